# baseline (speedup 1.0000x reference)
import functools

import jax
import jax.numpy as jnp
from jax.experimental import pallas as pl
from jax.experimental.pallas import tpu as pltpu


def _subpix_conv_kernel(
    x_ref,
    d_ref,
    m_ref,
    b_ref,
    o_ref,
    xu_ref,
    *, th0, H0, W0, Cin, Cout,
):
    W = 2 * W0
    t = pl.program_id(1)
    r0 = t * th0

    @pl.when(t == 0)
    def _build_upsampled():
        x2d = x_ref[0].astype(jnp.bfloat16).reshape(Cin * H0, W0)
        for kx in range(3):
            xu = jnp.dot(x2d, d_ref[kx], preferred_element_type=jnp.float32)
            xu_ref[kx, :, 1:H0 + 1, :] = xu.reshape(Cin, H0, W).astype(
                jnp.bfloat16)
            xu_ref[kx, :, 0:1, :] = xu_ref[kx, :, 1:2, :]
            xu_ref[kx, :, H0 + 1:H0 + 2, :] = xu_ref[kx, :, H0:H0 + 1, :]

    xw = xu_ref[:, :, pl.ds(r0, th0 + 8), :]
    x3 = xw.reshape(3 * Cin, (th0 + 8) * W)

    acc = None
    for dy in range(3):
        c = jnp.dot(m_ref[dy], x3[:, dy * W: (dy + th0) * W],
                    preferred_element_type=jnp.float32)
        acc = c if acc is None else acc + c
    y = acc + b_ref[...]

    o_ref[0] = y.reshape(2, Cout, th0 * W).astype(o_ref.dtype)


def kernel(x, weight, bias):
    B, Cin, H0, W0 = x.shape
    Cout = weight.shape[0]
    k = weight.shape[2]
    s = 2
    H, W = H0 * s, W0 * s
    assert k == 3

    th0 = 64 if H0 % 64 == 0 else H0
    n_h = H0 // th0

    f32 = jnp.float32
    R = jnp.zeros((2, 3, 3), f32)
    for a, d, e in [(0, 0, 0), (0, 1, 1), (0, 1, 2),
                    (1, 1, 0), (1, 1, 1), (1, 2, 2)]:
        R = R.at[a, d, e].set(1.0)

    Wc = jnp.einsum('ade,oieg->aodgi', R, weight.astype(f32))
    M2 = Wc.reshape(2 * Cout, 3, 3 * Cin).transpose(1, 0, 2).astype(jnp.bfloat16)
    bias2 = jnp.tile(bias.astype(f32), 2).reshape(2 * Cout, 1)

    q = jnp.clip(jnp.arange(W, dtype=jnp.int32)[None, :]
                 + jnp.arange(k, dtype=jnp.int32)[:, None] - 1, 0, W - 1) // s
    D3 = (q[:, None, :] ==
          jnp.arange(W0, dtype=jnp.int32)[None, :, None]).astype(jnp.bfloat16)

    body = functools.partial(
        _subpix_conv_kernel, th0=th0, H0=H0, W0=W0, Cin=Cin, Cout=Cout)

    out = pl.pallas_call(
        body,
        out_shape=jax.ShapeDtypeStruct((B, 2, Cout, H0 * W), x.dtype),
        grid=(B, n_h),
        in_specs=[
            pl.BlockSpec((1, Cin, H0, W0), lambda b, t: (b, 0, 0, 0)),
            pl.BlockSpec((3, W0, W), lambda b, t: (0, 0, 0)),
            pl.BlockSpec((3, 2 * Cout, 3 * Cin), lambda b, t: (0, 0, 0)),
            pl.BlockSpec((2 * Cout, 1), lambda b, t: (0, 0)),
        ],
        out_specs=pl.BlockSpec((1, 2, Cout, th0 * W), lambda b, t: (b, 0, 0, t)),
        scratch_shapes=[pltpu.VMEM((3, Cin, H0 + 8, W), jnp.bfloat16)],
        compiler_params=pltpu.CompilerParams(
            dimension_semantics=("parallel", "arbitrary"),
            vmem_limit_bytes=56 << 20,
        ),
    )(x, D3, M2, bias2)

    out = out.reshape(B, 2, Cout, H0, W).transpose(0, 2, 3, 1, 4)
    return out.reshape(B, Cout, H, W)


# speedup vs baseline: 1.5214x; 1.5214x over previous
import functools

import jax
import jax.numpy as jnp
from jax.experimental import pallas as pl
from jax.experimental.pallas import tpu as pltpu


def _subpix_conv_kernel(
    x_ref,
    d_ref,
    m_ref,
    b_ref,
    o_ref,
    xu_ref,
    *, th0, H0, W0, Cin, Cout,
):
    W = 2 * W0
    t = pl.program_id(1)
    r0 = t * th0

    @pl.when(t == 0)
    def _build_upsampled():
        xu = jnp.dot(x_ref[0].astype(jnp.bfloat16).reshape(Cin * H0, W0),
                     d_ref[...], preferred_element_type=jnp.float32)
        xu_ref[:, 1:H0 + 1, :] = xu.reshape(Cin, H0, W).astype(jnp.bfloat16)
        xu_ref[:, 0:1, :] = xu_ref[:, 1:2, :]
        xu_ref[:, H0 + 1:H0 + 2, :] = xu_ref[:, H0:H0 + 1, :]

    xw = xu_ref[:, pl.ds(r0, th0 + 8), :]
    left = jnp.concatenate([xw[:, :, :1], xw[:, :, :-1]], axis=2)
    right = jnp.concatenate([xw[:, :, 1:], xw[:, :, -1:]], axis=2)
    x3 = jnp.concatenate([left, xw, right], axis=0)
    x3 = x3.reshape(3 * Cin, (th0 + 8) * W)

    acc = None
    for dy in range(3):
        c = jnp.dot(m_ref[dy], x3[:, dy * W: (dy + th0) * W],
                    preferred_element_type=jnp.float32)
        acc = c if acc is None else acc + c
    y = acc + b_ref[...]

    z = y.reshape(2, Cout, th0, W)
    for a in range(2):
        o_ref[0, :, :, a, :] = z[a].astype(o_ref.dtype)


def kernel(x, weight, bias):
    B, Cin, H0, W0 = x.shape
    Cout = weight.shape[0]
    k = weight.shape[2]
    s = 2
    H, W = H0 * s, W0 * s
    assert k == 3

    th0 = 128 if H0 % 128 == 0 else (64 if H0 % 64 == 0 else H0)
    n_h = H0 // th0

    f32 = jnp.float32
    R = jnp.zeros((2, 3, 3), f32)
    for a, d, e in [(0, 0, 0), (0, 1, 1), (0, 1, 2),
                    (1, 1, 0), (1, 1, 1), (1, 2, 2)]:
        R = R.at[a, d, e].set(1.0)

    Wc = jnp.einsum('ade,oieg->aodgi', R, weight.astype(f32))
    M2 = Wc.reshape(2 * Cout, 3, 3 * Cin).transpose(1, 0, 2).astype(jnp.bfloat16)
    bias2 = jnp.tile(bias.astype(f32), 2).reshape(2 * Cout, 1)

    D = (jnp.arange(W, dtype=jnp.int32)[None, :] // s
         == jnp.arange(W0, dtype=jnp.int32)[:, None]).astype(jnp.bfloat16)

    body = functools.partial(
        _subpix_conv_kernel, th0=th0, H0=H0, W0=W0, Cin=Cin, Cout=Cout)

    out = pl.pallas_call(
        body,
        out_shape=jax.ShapeDtypeStruct((B, Cout, H0, 2, W), x.dtype),
        grid=(B, n_h),
        in_specs=[
            pl.BlockSpec((1, Cin, H0, W0), lambda b, t: (b, 0, 0, 0)),
            pl.BlockSpec((W0, W), lambda b, t: (0, 0)),
            pl.BlockSpec((3, 2 * Cout, 3 * Cin), lambda b, t: (0, 0, 0)),
            pl.BlockSpec((2 * Cout, 1), lambda b, t: (0, 0)),
        ],
        out_specs=pl.BlockSpec((1, Cout, th0, 2, W), lambda b, t: (b, 0, t, 0, 0)),
        scratch_shapes=[pltpu.VMEM((Cin, H0 + 8, W), jnp.bfloat16)],
        compiler_params=pltpu.CompilerParams(
            dimension_semantics=("parallel", "arbitrary"),
            vmem_limit_bytes=56 << 20,
        ),
    )(x, D, M2, bias2)

    return out.reshape(B, Cout, H, W)
